# Initial kernel scaffold; baseline (speedup 1.0000x reference)
import jax
import jax.numpy as jnp
from jax.experimental import pallas as pl


def kernel(x, edge_index, batch, W1, b1, Wmu, bmu, Wlv, blv):
    raise NotImplementedError("write your pallas kernel here")


# speedup vs baseline: 8.1489x; 8.1489x over previous
import functools

import jax
import jax.numpy as jnp
from jax import lax
from jax.experimental import pallas as pl
from jax.experimental.pallas import tpu as pltpu
from jax.experimental.pallas import tpu_sc as plsc

N = 10000
D = 128
DOUT = 64
E = 320000

NC = 2
NS = 16
NW = NC * NS

CH = 128
NCH = 80
NPH = 2
NCHP = NCH // NPH
EPT = NCH * CH
EPAD = EPT * NW
NPAD = 10240
RPT = NPAD // NS
DEGW = 16
DEGWIN = 8

_mesh = plsc.VectorSubcoreMesh(
    core_axis_name="c", subcore_axis_name="s", num_cores=NC, num_subcores=NS)

_Z16 = (16,)


def _fill(ref, rows, width, value):
    vec = jnp.full(_Z16, value, jnp.float32)

    def row(r, carry):
        for k in range(width // 16):
            ref[r, pl.ds(k * 16, 16)] = vec
        return carry

    lax.fori_loop(0, rows, row, 0)


def _deg_body(dst3, out_hbm, didx, ones, zb, table, s0, s1):
    c = lax.axis_index("c")
    s = lax.axis_index("s")
    w = c * NS + s
    _fill(ones, CH, DEGW, 1.0)
    _fill(zb, 64, DEGW, 0.0)
    r0 = s * RPT
    for k in range(RPT // 64):
        pltpu.sync_copy(zb, table.at[pl.ds(r0 + k * 64, 64)])
    plsc.subcore_barrier()
    pltpu.sync_copy(dst3.at[w], didx)

    sems = (s0, s1)

    def fire(jj, sem):
        pltpu.async_copy(ones, table.at[didx.at[jj]], sem, add=True)

    fire(0, s0)

    def body(j2, carry):
        for b in range(2):
            jj = j2 * 2 + b

            @pl.when(jj + 1 < NCH)
            def _():
                fire(jj + 1, sems[1 - b])

            pltpu.make_async_copy(
                ones, table.at[didx.at[jj]], sems[b]).wait()
        return carry

    lax.fori_loop(0, NCH // 2, body, 0)
    plsc.subcore_barrier()
    pltpu.sync_copy(table.at[pl.ds(r0, RPT)], out_hbm.at[c].at[pl.ds(r0, RPT)])


def _build_deg(interpret=False):
    return functools.partial(
        pl.kernel,
        out_type=jax.ShapeDtypeStruct((NC, NPAD, DEGW), jnp.float32),
        mesh=_mesh,
        scratch_types=[
            pltpu.VMEM((NCH, CH), jnp.int32),
            pltpu.VMEM((CH, DEGW), jnp.float32),
            pltpu.VMEM((64, DEGW), jnp.float32),
            pltpu.VMEM_SHARED((NPAD, DEGW), jnp.float32),
            pltpu.SemaphoreType.DMA,
            pltpu.SemaphoreType.DMA,
        ],
        interpret=interpret,
    )(_deg_body)


_deg = _build_deg()


def _prop_body(y_hbm, src4, dst4, out_hbm, sidx, didx, rb0, rb1, acc,
               sg0, sg1, ss0, ss1):
    c = lax.axis_index("c")
    s = lax.axis_index("s")
    w = c * NS + s
    _fill(rb0, CH, D, 0.0)
    r0 = s * RPT
    for k in range(RPT // CH):
        pltpu.sync_copy(rb0, acc.at[pl.ds(r0 + k * CH, CH)])
    plsc.subcore_barrier()

    rbs = (rb0, rb1)
    sgs = (sg0, sg1)
    sss = (ss0, ss1)

    for phase in range(NPH):
        pltpu.sync_copy(src4.at[w].at[phase], sidx)
        pltpu.sync_copy(dst4.at[w].at[phase], didx)

        pltpu.async_copy(y_hbm.at[sidx.at[0]], rb0, sg0)
        pltpu.async_copy(y_hbm.at[sidx.at[1]], rb1, sg1)

        def body(j2, carry):
            for b in range(2):
                jj = j2 * 2 + b
                pltpu.make_async_copy(
                    y_hbm.at[sidx.at[jj]], rbs[b], sgs[b]).wait()
                pltpu.async_copy(rbs[b], acc.at[didx.at[jj]], sss[b], add=True)
                pltpu.make_async_copy(
                    rbs[b], acc.at[didx.at[jj]], sss[b]).wait()

                @pl.when(jj + 2 < NCHP)
                def _():
                    pltpu.async_copy(y_hbm.at[sidx.at[jj + 2]], rbs[b], sgs[b])

            return carry

        lax.fori_loop(0, NCHP // 2, body, 0)

    plsc.subcore_barrier()
    pltpu.sync_copy(acc.at[pl.ds(r0, RPT)], out_hbm.at[c].at[pl.ds(r0, RPT)])


def _build_prop(interpret=False):
    return functools.partial(
        pl.kernel,
        out_type=jax.ShapeDtypeStruct((NC, NPAD, D), jnp.float32),
        mesh=_mesh,
        scratch_types=[
            pltpu.VMEM((NCHP, CH), jnp.int32),
            pltpu.VMEM((NCHP, CH), jnp.int32),
            pltpu.VMEM((CH, D), jnp.float32),
            pltpu.VMEM((CH, D), jnp.float32),
            pltpu.VMEM_SHARED((NPAD, D), jnp.float32),
            pltpu.SemaphoreType.DMA,
            pltpu.SemaphoreType.DMA,
            pltpu.SemaphoreType.DMA,
            pltpu.SemaphoreType.DMA,
        ],
        interpret=interpret,
    )(_prop_body)


_prop = _build_prop()


_R = 1000
_G = N // _R


def _dis(d0, d1):
    return lax.rsqrt(d0[...] + d1[...] + 1.0)


def _mm_scale_body(xb, wb, d0, d1, ob):
    ob[...] = _dis(d0, d1) * jnp.dot(
        xb[...], wb[...], preferred_element_type=jnp.float32)


def _mid_body(p0, p1, yb, d0, d1, wb, b1b, ob):
    dis = _dis(d0, d1)
    conv = dis * (p0[0] + p1[0] + yb[...]) + b1b[...]
    h = jnp.where(conv >= 0, conv, 0.01 * conv)
    ob[...] = dis * jnp.dot(h, wb[...], preferred_element_type=jnp.float32)


def _fin_body(q0, q1, yb, d0, d1, bm, bl, om, ol):
    z = _dis(d0, d1) * (q0[0] + q1[0] + yb[...])
    om[...] = z[:, :DOUT] + bm[...]
    ol[...] = z[:, DOUT:] + bl[...]


_row_spec = pl.BlockSpec((_R, D), lambda i: (i, 0))
_w_spec = pl.BlockSpec((D, D), lambda i: (0, 0))
_d_spec = pl.BlockSpec((_R, 1), lambda i: (i, 0))
_p0_spec = pl.BlockSpec((1, _R, D), lambda i: (0, i, 0))
_p1_spec = pl.BlockSpec((1, _R, D), lambda i: (1, i, 0))
_b_spec = pl.BlockSpec((1, D), lambda i: (0, 0))
_bh_spec = pl.BlockSpec((1, DOUT), lambda i: (0, 0))
_out_f32 = jax.ShapeDtypeStruct((N, D), jnp.float32)

_mm_scale = pl.pallas_call(
    _mm_scale_body, grid=(_G,),
    in_specs=[_row_spec, _w_spec, _d_spec, _d_spec],
    out_specs=_row_spec, out_shape=_out_f32)

_mid = pl.pallas_call(
    _mid_body, grid=(_G,),
    in_specs=[_p0_spec, _p1_spec, _row_spec, _d_spec, _d_spec, _w_spec, _b_spec],
    out_specs=_row_spec, out_shape=_out_f32)

_fin = pl.pallas_call(
    _fin_body, grid=(_G,),
    in_specs=[_p0_spec, _p1_spec, _row_spec, _d_spec, _d_spec, _bh_spec, _bh_spec],
    out_specs=[pl.BlockSpec((_R, DOUT), lambda i: (i, 0))] * 2,
    out_shape=[jax.ShapeDtypeStruct((N, DOUT), jnp.float32)] * 2)


def kernel(x, edge_index, batch, W1, b1, Wmu, bmu, Wlv, blv):
    src = edge_index[0]
    dst = edge_index[1]
    pad = EPAD - E
    src4 = jnp.concatenate(
        [src, jnp.zeros((pad,), src.dtype)]).reshape(NW, NPH, NCHP, CH)
    dst4 = jnp.concatenate(
        [dst, jnp.full((pad,), N, dst.dtype)]).reshape(NW, NPH, NCHP, CH)

    degp = _prop(jnp.ones((N, D), jnp.float32), src4, dst4)
    deg0 = degp[0, :N, 0:1]
    deg1 = degp[1, :N, 0:1]

    y1 = _mm_scale(x, W1, deg0, deg1)
    P = _prop(y1, src4, dst4)

    Wml = jnp.concatenate([Wmu, Wlv], axis=1)
    y2 = _mid(P, P, y1, deg0, deg1, Wml, b1.reshape(1, D))
    Q = _prop(y2, src4, dst4)

    mu, logvar = _fin(Q, Q, y2, deg0, deg1,
                      bmu.reshape(1, DOUT), blv.reshape(1, DOUT))
    return (mu, logvar)


# speedup vs baseline: 10.8129x; 1.3269x over previous
import functools

import jax
import jax.numpy as jnp
from jax import lax
from jax.experimental import pallas as pl
from jax.experimental.pallas import tpu as pltpu
from jax.experimental.pallas import tpu_sc as plsc

N = 10000
D = 128
DOUT = 64
E = 320000

NC = 2
NS = 16
NW = NC * NS

CH = 128
NCH = 80
NPH = 2
NCHP = NCH // NPH
EPT = NCH * CH
EPAD = EPT * NW
NPAD = 10240
RPT = NPAD // NS
DEGW = 16
DEGWIN = 8

_mesh = plsc.VectorSubcoreMesh(
    core_axis_name="c", subcore_axis_name="s", num_cores=NC, num_subcores=NS)

_Z16 = (16,)


def _fill(ref, rows, width, value):
    vec = jnp.full(_Z16, value, jnp.float32)

    def row(r, carry):
        for k in range(width // 16):
            ref[r, pl.ds(k * 16, 16)] = vec
        return carry

    lax.fori_loop(0, rows, row, 0)


def _deg_body(dst3, out_hbm, didx, ones, table, s0, s1, s2, s3):
    c = lax.axis_index("c")
    s = lax.axis_index("s")
    w = c * NS + s
    _fill(ones, CH, D, 0.0)
    r0 = s * RPT
    for k in range(RPT // CH):
        pltpu.sync_copy(ones, table.at[pl.ds(r0 + k * CH, CH)])
    _fill(ones, CH, D, 1.0)
    plsc.subcore_barrier()
    pltpu.sync_copy(dst3.at[w], didx)

    sems = (s0, s1, s2, s3)
    NSEM = len(sems)

    def fire(jj, sem):
        pltpu.async_copy(ones, table.at[didx.at[jj]], sem, add=True)

    def wait(jj, sem):
        pltpu.make_async_copy(ones, table.at[didx.at[jj]], sem).wait()

    for b in range(NSEM):
        fire(b, sems[b])

    def body(j4, carry):
        for b in range(NSEM):
            jj = j4 * NSEM + b
            wait(jj - NSEM, sems[b])
            fire(jj, sems[b])
        return carry

    lax.fori_loop(1, NCH // NSEM, body, 0)
    for b in range(NSEM):
        wait(NCH - NSEM + b, sems[b])
    plsc.subcore_barrier()
    pltpu.sync_copy(table.at[pl.ds(r0, RPT)], out_hbm.at[c].at[pl.ds(r0, RPT)])


def _build_deg(interpret=False):
    return functools.partial(
        pl.kernel,
        out_type=jax.ShapeDtypeStruct((NC, NPAD, D), jnp.float32),
        mesh=_mesh,
        scratch_types=[
            pltpu.VMEM((NCH, CH), jnp.int32),
            pltpu.VMEM((CH, D), jnp.float32),
            pltpu.VMEM_SHARED((NPAD, D), jnp.float32),
            pltpu.SemaphoreType.DMA,
            pltpu.SemaphoreType.DMA,
            pltpu.SemaphoreType.DMA,
            pltpu.SemaphoreType.DMA,
        ],
        interpret=interpret,
    )(_deg_body)


_deg = _build_deg()


def _prop_body(y_hbm, src4, dst4, out_hbm, sidx, didx, rb0, rb1, acc,
               sg0, sg1, ss0, ss1):
    c = lax.axis_index("c")
    s = lax.axis_index("s")
    w = c * NS + s
    _fill(rb0, CH, D, 0.0)
    r0 = s * RPT
    for k in range(RPT // CH):
        pltpu.sync_copy(rb0, acc.at[pl.ds(r0 + k * CH, CH)])
    plsc.subcore_barrier()

    rbs = (rb0, rb1)
    sgs = (sg0, sg1)
    sss = (ss0, ss1)

    for phase in range(NPH):
        pltpu.sync_copy(src4.at[w].at[phase], sidx)
        pltpu.sync_copy(dst4.at[w].at[phase], didx)

        pltpu.async_copy(y_hbm.at[sidx.at[0]], rb0, sg0)
        pltpu.async_copy(y_hbm.at[sidx.at[1]], rb1, sg1)

        def body(j2, carry):
            for b in range(2):
                jj = j2 * 2 + b
                pltpu.make_async_copy(
                    y_hbm.at[sidx.at[jj]], rbs[b], sgs[b]).wait()
                pltpu.async_copy(rbs[b], acc.at[didx.at[jj]], sss[b], add=True)
                pltpu.make_async_copy(
                    rbs[b], acc.at[didx.at[jj]], sss[b]).wait()

                @pl.when(jj + 2 < NCHP)
                def _():
                    pltpu.async_copy(y_hbm.at[sidx.at[jj + 2]], rbs[b], sgs[b])

            return carry

        lax.fori_loop(0, NCHP // 2, body, 0)

    plsc.subcore_barrier()
    pltpu.sync_copy(acc.at[pl.ds(r0, RPT)], out_hbm.at[c].at[pl.ds(r0, RPT)])


def _build_prop(interpret=False):
    return functools.partial(
        pl.kernel,
        out_type=jax.ShapeDtypeStruct((NC, NPAD, D), jnp.float32),
        mesh=_mesh,
        scratch_types=[
            pltpu.VMEM((NCHP, CH), jnp.int32),
            pltpu.VMEM((NCHP, CH), jnp.int32),
            pltpu.VMEM((CH, D), jnp.float32),
            pltpu.VMEM((CH, D), jnp.float32),
            pltpu.VMEM_SHARED((NPAD, D), jnp.float32),
            pltpu.SemaphoreType.DMA,
            pltpu.SemaphoreType.DMA,
            pltpu.SemaphoreType.DMA,
            pltpu.SemaphoreType.DMA,
        ],
        interpret=interpret,
    )(_prop_body)


_prop = _build_prop()


_R = 1000
_G = N // _R


def _dis(d0, d1):
    return lax.rsqrt(d0[...] + d1[...] + 1.0)


def _mm_scale_body(xb, wb, d0, d1, ob):
    ob[...] = _dis(d0, d1) * jnp.dot(
        xb[...], wb[...], preferred_element_type=jnp.float32)


def _mid_body(p0, p1, yb, d0, d1, wb, b1b, ob):
    dis = _dis(d0, d1)
    conv = dis * (p0[0] + p1[0] + yb[...]) + b1b[...]
    h = jnp.where(conv >= 0, conv, 0.01 * conv)
    ob[...] = dis * jnp.dot(h, wb[...], preferred_element_type=jnp.float32)


def _fin_body(q0, q1, yb, d0, d1, bm, bl, om, ol):
    z = _dis(d0, d1) * (q0[0] + q1[0] + yb[...])
    om[...] = z[:, :DOUT] + bm[...]
    ol[...] = z[:, DOUT:] + bl[...]


_row_spec = pl.BlockSpec((_R, D), lambda i: (i, 0))
_w_spec = pl.BlockSpec((D, D), lambda i: (0, 0))
_d_spec = pl.BlockSpec((_R, 1), lambda i: (i, 0))
_p0_spec = pl.BlockSpec((1, _R, D), lambda i: (0, i, 0))
_p1_spec = pl.BlockSpec((1, _R, D), lambda i: (1, i, 0))
_b_spec = pl.BlockSpec((1, D), lambda i: (0, 0))
_bh_spec = pl.BlockSpec((1, DOUT), lambda i: (0, 0))
_out_f32 = jax.ShapeDtypeStruct((N, D), jnp.float32)

_mm_scale = pl.pallas_call(
    _mm_scale_body, grid=(_G,),
    in_specs=[_row_spec, _w_spec, _d_spec, _d_spec],
    out_specs=_row_spec, out_shape=_out_f32)

_mid = pl.pallas_call(
    _mid_body, grid=(_G,),
    in_specs=[_p0_spec, _p1_spec, _row_spec, _d_spec, _d_spec, _w_spec, _b_spec],
    out_specs=_row_spec, out_shape=_out_f32)

_fin = pl.pallas_call(
    _fin_body, grid=(_G,),
    in_specs=[_p0_spec, _p1_spec, _row_spec, _d_spec, _d_spec, _bh_spec, _bh_spec],
    out_specs=[pl.BlockSpec((_R, DOUT), lambda i: (i, 0))] * 2,
    out_shape=[jax.ShapeDtypeStruct((N, DOUT), jnp.float32)] * 2)


def kernel(x, edge_index, batch, W1, b1, Wmu, bmu, Wlv, blv):
    src = edge_index[0]
    dst = edge_index[1]
    pad = EPAD - E
    src4 = jnp.concatenate(
        [src, jnp.zeros((pad,), src.dtype)]).reshape(NW, NPH, NCHP, CH)
    dst4 = jnp.concatenate(
        [dst, jnp.full((pad,), N, dst.dtype)]).reshape(NW, NPH, NCHP, CH)

    degp = _deg(dst4.reshape(NW, NCH, CH))
    deg0 = degp[0, :N, 0:1]
    deg1 = degp[1, :N, 0:1]

    y1 = _mm_scale(x, W1, deg0, deg1)
    P = _prop(y1, src4, dst4)

    Wml = jnp.concatenate([Wmu, Wlv], axis=1)
    y2 = _mid(P, P, y1, deg0, deg1, Wml, b1.reshape(1, D))
    Q = _prop(y2, src4, dst4)

    mu, logvar = _fin(Q, Q, y2, deg0, deg1,
                      bmu.reshape(1, DOUT), blv.reshape(1, DOUT))
    return (mu, logvar)
